# baseline (speedup 1.0000x reference)
import functools

import jax
import jax.numpy as jnp
from jax import lax
from jax.experimental import pallas as pl
from jax.experimental.pallas import tpu as pltpu
from jax.experimental.pallas import tpu_sc as plsc

_N = 10000
_E = 320000
_G = 64
_DIN = 128
_DH = 32
_NCLS = 2

_NC = 2
_NS = 16
_NW = _NC * _NS
_EPW = _E // _NW
_CH = 80
_NCHUNK = _EPW // _CH
_NPAD = 10240
_RPS = _NPAD // _NS
_DEGW = 16


def _sc_mesh():
    return plsc.VectorSubcoreMesh(core_axis_name="c", subcore_axis_name="s",
                                  num_cores=_NC, num_subcores=_NS)


_SC_PARAMS = pltpu.CompilerParams(use_tc_tiling_on_sc=False)


def _edge_pass(u_hbm, src_v, dst_v, rows, acc, gsem, ssem):
    def gather(k, b):
        pltpu.async_copy(u_hbm.at[src_v.at[k]], rows.at[b], gsem[b])

    def gwait(b):
        pltpu.make_async_copy(u_hbm.at[src_v.at[0]], rows.at[b], gsem[b]).wait()

    def scat(k, b):
        pltpu.async_copy(rows.at[b], acc.at[dst_v.at[k]], ssem, add=True)

    def swait(b):
        pltpu.make_async_copy(rows.at[b], acc.at[dst_v.at[0]], ssem).wait()

    gather(0, 0)
    gather(1, 1)
    gather(2, 2)
    gather(3, 3)

    def body(j, carry):
        for p in range(4):
            k = 4 * j + p
            gwait(p)
            scat(k, p)
            swait(p)
            gather(k + 4, p)
        return carry

    lax.fori_loop(0, (_NCHUNK - 5) // 4, body, 0)

    gwait(0)
    scat(_NCHUNK - 5, 0)
    swait(0)
    gather(_NCHUNK - 1, 0)
    for p in range(1, 4):
        gwait(p)
        scat(_NCHUNK - 5 + p, p)
        swait(p)
    gwait(0)
    scat(_NCHUNK - 1, 0)
    swait(0)


@functools.partial(
    pl.kernel,
    out_type=jax.ShapeDtypeStruct((_NC, _NPAD, _DEGW), jnp.float32),
    mesh=_sc_mesh(),
    scratch_types=[
        pltpu.VMEM((_NCHUNK, _CH), jnp.int32),
        pltpu.VMEM((_CH, _DEGW), jnp.float32),
        pltpu.VMEM((_RPS, _DEGW), jnp.float32),
        pltpu.VMEM_SHARED((_NPAD, _DEGW), jnp.float32),
        pltpu.SemaphoreType.DMA,
    ],
    compiler_params=_SC_PARAMS,
)
def _deg_kernel(dst_hbm, ones_hbm, out_hbm, dst_v, ones_v, zdeg, acc, sem):
    cid = lax.axis_index("c")
    sid = lax.axis_index("s")
    wid = sid * _NC + cid
    r0 = sid * _RPS
    zeros16 = jnp.zeros((16,), jnp.float32)

    def zb(i, carry):
        zdeg[i, pl.ds(0, 16)] = zeros16
        return carry

    lax.fori_loop(0, _RPS, zb, 0)
    pltpu.sync_copy(zdeg, acc.at[pl.ds(r0, _RPS)])
    pltpu.sync_copy(ones_hbm, ones_v)
    pltpu.sync_copy(dst_hbm.at[wid], dst_v)
    plsc.subcore_barrier()

    def fire(k, carry):
        pltpu.async_copy(ones_v, acc.at[dst_v.at[k]], sem, add=True)
        return carry

    lax.fori_loop(0, _NCHUNK, fire, 0)

    def drain(k, carry):
        pltpu.make_async_copy(ones_v, acc.at[dst_v.at[0]], sem).wait()
        return carry

    lax.fori_loop(0, _NCHUNK, drain, 0)
    plsc.subcore_barrier()
    pltpu.sync_copy(acc.at[pl.ds(r0, _RPS)], out_hbm.at[cid, pl.ds(r0, _RPS)])


@functools.partial(
    pl.kernel,
    out_type=(jax.ShapeDtypeStruct((_NC, _NPAD, _DH), jnp.float32),
              jax.ShapeDtypeStruct((_NPAD, _DH), jnp.float32)),
    mesh=_sc_mesh(),
    scratch_types=[
        pltpu.VMEM((_NCHUNK, _CH), jnp.int32),
        pltpu.VMEM((_NCHUNK, _CH), jnp.int32),
        pltpu.VMEM((2, _RPS, _DEGW), jnp.float32),
        pltpu.VMEM((_RPS, _DH), jnp.float32),
        pltpu.VMEM((_RPS // 2, _DH), jnp.float32),
        pltpu.VMEM((4, _CH, _DH), jnp.float32),
        pltpu.VMEM_SHARED((_NPAD, _DH), jnp.float32),
        pltpu.SemaphoreType.DMA,
        pltpu.SemaphoreType.DMA,
        pltpu.SemaphoreType.DMA,
        pltpu.SemaphoreType.DMA,
        pltpu.SemaphoreType.DMA,
    ],
    compiler_params=_SC_PARAMS,
)
def _layer1_kernel(p1_hbm, src_hbm, dst_hbm, degp_hbm,
                   out_hbm, u1_hbm,
                   src_v, dst_v, degred, p1loc, zacc, rows, acc,
                   g0, g1, g2, g3, ssem):
    cid = lax.axis_index("c")
    sid = lax.axis_index("s")
    wid = sid * _NC + cid
    r0 = sid * _RPS

    pltpu.sync_copy(src_hbm.at[wid], src_v)
    pltpu.sync_copy(dst_hbm.at[wid], dst_v)
    pltpu.sync_copy(degp_hbm.at[0, pl.ds(r0, _RPS)], degred.at[0])
    pltpu.sync_copy(degp_hbm.at[1, pl.ds(r0, _RPS)], degred.at[1])
    pltpu.sync_copy(p1_hbm.at[pl.ds(r0, _RPS)], p1loc)

    zeros16 = jnp.zeros((16,), jnp.float32)

    def zb1(i, carry):
        zacc[i, pl.ds(0, 16)] = zeros16
        zacc[i, pl.ds(16, 16)] = zeros16
        return carry

    lax.fori_loop(0, _RPS // 2, zb1, 0)
    pltpu.sync_copy(zacc, acc.at[pl.ds(r0, _RPS // 2)])
    pltpu.sync_copy(zacc, acc.at[pl.ds(r0 + _RPS // 2, _RPS // 2)])

    def dbody(i, carry):
        d = degred[0, i, pl.ds(0, _DEGW)] + degred[1, i, pl.ds(0, _DEGW)] + 1.0
        y = jnp.where(d >= 4.0, 0.35355339, 0.70710678)
        y = jnp.where(d >= 16.0, 0.17677670, y)
        y = jnp.where(d >= 64.0, 0.08838835, y)
        y = jnp.where(d >= 256.0, 0.04419417, y)
        y = jnp.where(d >= 1024.0, 0.02209709, y)
        y = jnp.where(d >= 4096.0, 0.01104854, y)
        y = jnp.where(d >= 16384.0, 0.00552427, y)
        y = jnp.where(d >= 65536.0, 0.00276214, y)
        y = jnp.where(d >= 262144.0, 0.00138107, y)
        for _ in range(5):
            y = y * (1.5 - 0.5 * d * y * y)
        p1loc[i, pl.ds(0, 16)] = p1loc[i, pl.ds(0, 16)] * y
        p1loc[i, pl.ds(16, 16)] = p1loc[i, pl.ds(16, 16)] * y
        return carry

    lax.fori_loop(0, _RPS, dbody, 0)

    pltpu.sync_copy(p1loc, u1_hbm.at[pl.ds(r0, _RPS)])

    plsc.subcore_barrier()

    _edge_pass(u1_hbm, src_v, dst_v, rows, acc, (g0, g1, g2, g3), ssem)

    plsc.subcore_barrier()
    pltpu.sync_copy(acc.at[pl.ds(r0, _RPS)], out_hbm.at[cid, pl.ds(r0, _RPS)])


@functools.partial(
    pl.kernel,
    out_type=jax.ShapeDtypeStruct((_NC, _NPAD, _DH), jnp.float32),
    mesh=_sc_mesh(),
    scratch_types=[
        pltpu.VMEM((_NCHUNK, _CH), jnp.int32),
        pltpu.VMEM((_NCHUNK, _CH), jnp.int32),
        pltpu.VMEM((4, _CH, _DH), jnp.float32),
        pltpu.VMEM_SHARED((_NPAD, _DH), jnp.float32),
        pltpu.SemaphoreType.DMA,
        pltpu.SemaphoreType.DMA,
        pltpu.SemaphoreType.DMA,
        pltpu.SemaphoreType.DMA,
        pltpu.SemaphoreType.DMA,
    ],
    compiler_params=_SC_PARAMS,
)
def _scatter_kernel(u_hbm, src_hbm, dst_hbm, zeros_hbm, out_hbm,
                    src_v, dst_v, rows, acc, g0, g1, g2, g3, ssem):
    cid = lax.axis_index("c")
    sid = lax.axis_index("s")
    wid = sid * _NC + cid
    r0 = sid * _RPS

    pltpu.sync_copy(src_hbm.at[wid], src_v)
    pltpu.sync_copy(dst_hbm.at[wid], dst_v)

    @pl.when(cid == 0)
    def _():
        pltpu.sync_copy(u_hbm.at[pl.ds(r0, _RPS)], acc.at[pl.ds(r0, _RPS)])

    @pl.when(cid != 0)
    def _():
        pltpu.sync_copy(zeros_hbm.at[pl.ds(r0, _RPS)], acc.at[pl.ds(r0, _RPS)])

    plsc.subcore_barrier()
    _edge_pass(u_hbm, src_v, dst_v, rows, acc, (g0, g1, g2, g3), ssem)
    plsc.subcore_barrier()
    pltpu.sync_copy(acc.at[pl.ds(r0, _RPS)], out_hbm.at[cid, pl.ds(r0, _RPS)])


def _mm1(x, W1):
    def body(x_ref, w_ref, p_ref):
        p_ref[:_N, :] = jnp.dot(x_ref[...], w_ref[...],
                                preferred_element_type=jnp.float32)
        p_ref[_N:, :] = jnp.zeros((_NPAD - _N, _DH), jnp.float32)

    return pl.pallas_call(
        body,
        out_shape=jax.ShapeDtypeStruct((_NPAD, _DH), jnp.float32),
    )(x, W1)


def _mm2(s1p, p1, degp, b1, W2):
    def body(s_ref, p_ref, d_ref, b_ref, w_ref, u_ref, di_ref):
        deg = d_ref[0, :_N, :1] + d_ref[1, :_N, :1] + 1.0
        di = lax.rsqrt(deg)
        di_ref[...] = di
        s = s_ref[0, :_N, :] + s_ref[1, :_N, :]
        p1n = p_ref[:_N, :]
        h1 = jnp.maximum(di * s + di * di * p1n + b_ref[...], 0.0)
        u_ref[:_N, :] = jnp.dot(h1, w_ref[...],
                                preferred_element_type=jnp.float32) * di
        u_ref[_N:, :] = jnp.zeros((_NPAD - _N, _DH), jnp.float32)

    return pl.pallas_call(
        body,
        out_shape=(jax.ShapeDtypeStruct((_NPAD, _DH), jnp.float32),
                   jax.ShapeDtypeStruct((_N, 1), jnp.float32)),
    )(s1p, p1, degp, b1, W2)


def _pool(s2p, dinv_col, b2, batch2d, Wl, bl):
    def body(s_ref, di_ref, b_ref, bat_ref, wl_ref, bl_ref, o_ref):
        di = di_ref[...]
        s = s_ref[0, :_N, :] + s_ref[1, :_N, :]
        h2 = jnp.maximum(s * di + b_ref[...], 0.0)
        gids = lax.broadcasted_iota(jnp.int32, (_G, _N), 0)
        onehot = (bat_ref[...] == gids).astype(jnp.float32)
        summed = jnp.dot(onehot, h2, preferred_element_type=jnp.float32)
        cnt = jnp.sum(onehot, axis=1, keepdims=True)
        pooled = summed / jnp.maximum(cnt, 1.0)
        o_ref[...] = jnp.dot(pooled, wl_ref[...],
                             preferred_element_type=jnp.float32) + bl_ref[...]

    return pl.pallas_call(
        body,
        out_shape=jax.ShapeDtypeStruct((_G, _NCLS), jnp.float32),
    )(s2p, dinv_col, b2, batch2d, Wl, bl)


def kernel(x, edge_index, batch, W1, b1, W2, b2, Wl, bl):
    src = edge_index[0].reshape(_NW, _NCHUNK, _CH)
    dst = edge_index[1].reshape(_NW, _NCHUNK, _CH)
    zeros_u = jnp.zeros((_NPAD, _DH), jnp.float32)
    ones_rows = jnp.ones((_CH, _DEGW), jnp.float32)

    degp = _deg_kernel(dst, ones_rows)
    p1 = _mm1(x, W1)
    s1p, _u1 = _layer1_kernel(p1, src, dst, degp)
    u2, dinv_col = _mm2(s1p, p1, degp, b1.reshape(1, _DH), W2)
    s2p = _scatter_kernel(u2, src, dst, zeros_u)
    out = _pool(s2p, dinv_col, b2.reshape(1, _DH), batch.reshape(1, _N),
                Wl, bl.reshape(1, _NCLS))
    return out


# speedup vs baseline: 1.1469x; 1.1469x over previous
import functools

import jax
import jax.numpy as jnp
from jax import lax
from jax.experimental import pallas as pl
from jax.experimental.pallas import tpu as pltpu
from jax.experimental.pallas import tpu_sc as plsc

_N = 10000
_E = 320000
_G = 64
_DIN = 128
_DH = 32
_NCLS = 2

_NC = 2
_NS = 16
_NW = _NC * _NS
_EPW = _E // _NW
_CH = 125
_NCHUNK = _EPW // _CH
_NPAD = 10240
_RPS = _NPAD // _NS
_DEGW = 8


def _sc_mesh():
    return plsc.VectorSubcoreMesh(core_axis_name="c", subcore_axis_name="s",
                                  num_cores=_NC, num_subcores=_NS)


_SC_PARAMS = pltpu.CompilerParams(use_tc_tiling_on_sc=False)


@functools.partial(
    pl.kernel,
    out_type=jax.ShapeDtypeStruct((_NC, _NPAD, _DEGW), jnp.float32),
    mesh=_sc_mesh(),
    scratch_types=[
        pltpu.VMEM((_NCHUNK, _CH), jnp.int32),
        pltpu.VMEM((_CH, _DEGW), jnp.float32),
        pltpu.VMEM_SHARED((_NPAD, _DEGW), jnp.float32),
        pltpu.SemaphoreType.DMA,
    ],
    compiler_params=_SC_PARAMS,
)
def _deg_kernel(dst_hbm, ones_hbm, zeros_hbm, out_hbm, dst_v, ones_v, acc, sem):
    cid = lax.axis_index("c")
    sid = lax.axis_index("s")
    wid = sid * _NC + cid
    r0 = sid * _RPS
    pltpu.sync_copy(zeros_hbm.at[pl.ds(r0, _RPS)], acc.at[pl.ds(r0, _RPS)])
    pltpu.sync_copy(ones_hbm, ones_v)
    pltpu.sync_copy(dst_hbm.at[wid], dst_v)
    plsc.subcore_barrier()

    def fire(k, carry):
        pltpu.async_copy(ones_v, acc.at[dst_v.at[k]], sem, add=True)
        return carry

    lax.fori_loop(0, _NCHUNK, fire, 0)

    def drain(k, carry):
        pltpu.make_async_copy(ones_v, acc.at[dst_v.at[0]], sem).wait()
        return carry

    lax.fori_loop(0, _NCHUNK, drain, 0)
    plsc.subcore_barrier()
    pltpu.sync_copy(acc.at[pl.ds(r0, _RPS)], out_hbm.at[cid, pl.ds(r0, _RPS)])


@functools.partial(
    pl.kernel,
    out_type=jax.ShapeDtypeStruct((_NC, _NPAD, _DH), jnp.float32),
    mesh=_sc_mesh(),
    scratch_types=[
        pltpu.VMEM((_NCHUNK, _CH), jnp.int32),
        pltpu.VMEM((_NCHUNK, _CH), jnp.int32),
        pltpu.VMEM((4, _CH, _DH), jnp.float32),
        pltpu.VMEM_SHARED((_NPAD, _DH), jnp.float32),
        pltpu.SemaphoreType.DMA,
        pltpu.SemaphoreType.DMA,
        pltpu.SemaphoreType.DMA,
        pltpu.SemaphoreType.DMA,
        pltpu.SemaphoreType.DMA,
        pltpu.SemaphoreType.DMA,
    ],
    compiler_params=_SC_PARAMS,
)
def _scatter_kernel(u_hbm, src_hbm, dst_hbm, zeros_hbm, out_hbm,
                    src_v, dst_v, rows, acc, g0, g1, g2, g3, s0, s1):
    cid = lax.axis_index("c")
    sid = lax.axis_index("s")
    wid = sid * _NC + cid
    r0 = sid * _RPS

    pltpu.sync_copy(src_hbm.at[wid], src_v)
    pltpu.sync_copy(dst_hbm.at[wid], dst_v)

    @pl.when(cid == 0)
    def _():
        pltpu.sync_copy(u_hbm.at[pl.ds(r0, _RPS)], acc.at[pl.ds(r0, _RPS)])

    @pl.when(cid != 0)
    def _():
        pltpu.sync_copy(zeros_hbm.at[pl.ds(r0, _RPS)], acc.at[pl.ds(r0, _RPS)])

    plsc.subcore_barrier()

    gsem = (g0, g1, g2, g3)
    ssem = (s0, s1)

    def gather(k, b):
        pltpu.async_copy(u_hbm.at[src_v.at[k]], rows.at[b], gsem[b])

    def gwait(b):
        pltpu.make_async_copy(u_hbm.at[src_v.at[0]], rows.at[b], gsem[b]).wait()

    def scat(k, b, q):
        pltpu.async_copy(rows.at[b], acc.at[dst_v.at[k]], ssem[q], add=True)

    def swait(q):
        pltpu.make_async_copy(rows.at[0], acc.at[dst_v.at[0]], ssem[q]).wait()

    gather(0, 0)
    gather(1, 1)
    gather(2, 2)
    gwait(0)
    scat(0, 0, 0)
    gather(3, 3)

    def body(j, carry):
        for i in range(4):
            k = 4 * j + 1 + i
            p = (1 + i) % 4
            gwait(p)
            scat(k, p, (1 + i) % 2)
            swait(i % 2)
            gather(k + 3, i % 4)
        return carry

    lax.fori_loop(0, (_NCHUNK - 4) // 4, body, 0)

    gwait(1)
    scat(_NCHUNK - 3, 1, 1)
    swait(0)
    gwait(2)
    scat(_NCHUNK - 2, 2, 0)
    swait(1)
    gwait(3)
    scat(_NCHUNK - 1, 3, 1)
    swait(0)
    swait(1)

    plsc.subcore_barrier()
    pltpu.sync_copy(acc.at[pl.ds(r0, _RPS)], out_hbm.at[cid, pl.ds(r0, _RPS)])


def _mm1(x, W1, degp):
    def body(x_ref, w_ref, d_ref, u_ref, di_ref):
        p1 = jnp.dot(x_ref[...], w_ref[...],
                     preferred_element_type=jnp.float32)
        d = d_ref[0] + d_ref[1]
        deg = d[:_N, :1] + 1.0
        dinv = lax.rsqrt(deg)
        di_ref[...] = dinv
        u_ref[:_N, :] = p1 * dinv
        u_ref[_N:, :] = jnp.zeros((_NPAD - _N, _DH), jnp.float32)

    return pl.pallas_call(
        body,
        out_shape=(jax.ShapeDtypeStruct((_NPAD, _DH), jnp.float32),
                   jax.ShapeDtypeStruct((_N, 1), jnp.float32)),
    )(x, W1, degp)


def _mm2(s1p, dinv, b1, W2):
    def body(s_ref, di_ref, b_ref, w_ref, u_ref):
        s = s_ref[0, :_N, :] + s_ref[1, :_N, :]
        di = di_ref[...]
        h1 = jnp.maximum(s * di + b_ref[...], 0.0)
        u_ref[:_N, :] = jnp.dot(h1, w_ref[...],
                                preferred_element_type=jnp.float32) * di
        u_ref[_N:, :] = jnp.zeros((_NPAD - _N, _DH), jnp.float32)

    return pl.pallas_call(
        body,
        out_shape=jax.ShapeDtypeStruct((_NPAD, _DH), jnp.float32),
    )(s1p, dinv, b1, W2)


def _pool(s2p, dinv, b2, batch2d, Wl, bl):
    def body(s_ref, di_ref, b_ref, bat_ref, wl_ref, bl_ref, o_ref):
        s = s_ref[0, :_N, :] + s_ref[1, :_N, :]
        h2 = jnp.maximum(s * di_ref[...] + b_ref[...], 0.0)
        gids = lax.broadcasted_iota(jnp.int32, (_G, _N), 0)
        onehot = (bat_ref[...] == gids).astype(jnp.float32)
        summed = jnp.dot(onehot, h2, preferred_element_type=jnp.float32)
        cnt = jnp.sum(onehot, axis=1, keepdims=True)
        pooled = summed / jnp.maximum(cnt, 1.0)
        o_ref[...] = jnp.dot(pooled, wl_ref[...],
                             preferred_element_type=jnp.float32) + bl_ref[...]

    return pl.pallas_call(
        body,
        out_shape=jax.ShapeDtypeStruct((_G, _NCLS), jnp.float32),
    )(s2p, dinv, b2, batch2d, Wl, bl)


def kernel(x, edge_index, batch, W1, b1, W2, b2, Wl, bl):
    src = edge_index[0].reshape(_NW, _NCHUNK, _CH)
    dst = edge_index[1].reshape(_NW, _NCHUNK, _CH)
    ones_rows = jnp.ones((_CH, _DEGW), jnp.float32)
    zeros_deg = jnp.zeros((_NPAD, _DEGW), jnp.float32)
    zeros_u = jnp.zeros((_NPAD, _DH), jnp.float32)

    degp = _deg_kernel(dst, ones_rows, zeros_deg)
    u1, dinv = _mm1(x, W1, degp)
    s1p = _scatter_kernel(u1, src, dst, zeros_u)
    u2 = _mm2(s1p, dinv, b1.reshape(1, _DH), W2)
    s2p = _scatter_kernel(u2, src, dst, zeros_u)
    out = _pool(s2p, dinv, b2.reshape(1, _DH), batch.reshape(1, _N),
                Wl, bl.reshape(1, _NCLS))
    return out


# speedup vs baseline: 1.1700x; 1.0202x over previous
import functools

import jax
import jax.numpy as jnp
from jax import lax
from jax.experimental import pallas as pl
from jax.experimental.pallas import tpu as pltpu
from jax.experimental.pallas import tpu_sc as plsc

_N = 10000
_E = 320000
_G = 64
_DIN = 128
_DH = 32
_NCLS = 2

_NC = 2
_NS = 16
_NW = _NC * _NS
_EPW = _E // _NW
_CH = 125
_NCHUNK = _EPW // _CH
_NPAD = 10240
_RPS = _NPAD // _NS
_DEGW = 8


def _sc_mesh():
    return plsc.VectorSubcoreMesh(core_axis_name="c", subcore_axis_name="s",
                                  num_cores=_NC, num_subcores=_NS)


_SC_PARAMS = pltpu.CompilerParams(use_tc_tiling_on_sc=False)


@functools.partial(
    pl.kernel,
    out_type=jax.ShapeDtypeStruct((_NC, _NPAD, _DEGW), jnp.float32),
    mesh=_sc_mesh(),
    scratch_types=[
        pltpu.VMEM((_NCHUNK, _CH), jnp.int32),
        pltpu.VMEM((_CH, _DEGW), jnp.float32),
        pltpu.VMEM_SHARED((_NPAD, _DEGW), jnp.float32),
        pltpu.SemaphoreType.DMA,
    ],
    compiler_params=_SC_PARAMS,
)
def _deg_kernel(dst_hbm, ones_hbm, zeros_hbm, out_hbm, dst_v, ones_v, acc, sem):
    cid = lax.axis_index("c")
    sid = lax.axis_index("s")
    wid = sid * _NC + cid
    r0 = sid * _RPS
    pltpu.sync_copy(zeros_hbm.at[pl.ds(r0, _RPS)], acc.at[pl.ds(r0, _RPS)])
    pltpu.sync_copy(ones_hbm, ones_v)
    pltpu.sync_copy(dst_hbm.at[wid], dst_v)
    plsc.subcore_barrier()

    def fire(k, carry):
        pltpu.async_copy(ones_v, acc.at[dst_v.at[k]], sem, add=True)
        return carry

    lax.fori_loop(0, _NCHUNK, fire, 0)

    def drain(k, carry):
        pltpu.make_async_copy(ones_v, acc.at[dst_v.at[0]], sem).wait()
        return carry

    lax.fori_loop(0, _NCHUNK, drain, 0)
    plsc.subcore_barrier()
    pltpu.sync_copy(acc.at[pl.ds(r0, _RPS)], out_hbm.at[cid, pl.ds(r0, _RPS)])


@functools.partial(
    pl.kernel,
    out_type=jax.ShapeDtypeStruct((_NC, _NPAD, _DH), jnp.float32),
    mesh=_sc_mesh(),
    scratch_types=[
        pltpu.VMEM((_NCHUNK, _CH), jnp.int32),
        pltpu.VMEM((_NCHUNK, _CH), jnp.int32),
        pltpu.VMEM((4, _CH, _DH), jnp.float32),
        pltpu.VMEM_SHARED((_NPAD, _DH), jnp.float32),
        pltpu.SemaphoreType.DMA,
        pltpu.SemaphoreType.DMA,
        pltpu.SemaphoreType.DMA,
        pltpu.SemaphoreType.DMA,
        pltpu.SemaphoreType.DMA,
    ],
    compiler_params=_SC_PARAMS,
)
def _scatter_kernel(u_hbm, src_hbm, dst_hbm, zeros_hbm, out_hbm,
                    src_v, dst_v, rows, acc, g0, g1, g2, g3, ssem):
    cid = lax.axis_index("c")
    sid = lax.axis_index("s")
    wid = sid * _NC + cid
    r0 = sid * _RPS

    pltpu.sync_copy(src_hbm.at[wid], src_v)
    pltpu.sync_copy(dst_hbm.at[wid], dst_v)

    @pl.when(cid == 0)
    def _():
        pltpu.sync_copy(u_hbm.at[pl.ds(r0, _RPS)], acc.at[pl.ds(r0, _RPS)])

    @pl.when(cid != 0)
    def _():
        pltpu.sync_copy(zeros_hbm.at[pl.ds(r0, _RPS)], acc.at[pl.ds(r0, _RPS)])

    plsc.subcore_barrier()

    gsem = (g0, g1, g2, g3)

    def gather(k, b):
        pltpu.async_copy(u_hbm.at[src_v.at[k]], rows.at[b], gsem[b])

    def gwait(b):
        pltpu.make_async_copy(u_hbm.at[src_v.at[0]], rows.at[b], gsem[b]).wait()

    def scat(k, b):
        pltpu.async_copy(rows.at[b], acc.at[dst_v.at[k]], ssem, add=True)

    def swait(b):
        pltpu.make_async_copy(rows.at[b], acc.at[dst_v.at[0]], ssem).wait()

    gather(0, 0)
    gather(1, 1)
    gather(2, 2)
    gather(3, 3)

    def body(j, carry):
        for p in range(4):
            k = 4 * j + p
            gwait(p)
            scat(k, p)
            swait(p)
            gather(k + 4, p)
        return carry

    lax.fori_loop(0, _NCHUNK // 4 - 1, body, 0)

    for p in range(4):
        gwait(p)
        scat(_NCHUNK - 4 + p, p)
        swait(p)

    plsc.subcore_barrier()
    pltpu.sync_copy(acc.at[pl.ds(r0, _RPS)], out_hbm.at[cid, pl.ds(r0, _RPS)])


def _mm1(x, W1, degp):
    def body(x_ref, w_ref, d_ref, u_ref, di_ref):
        p1 = jnp.dot(x_ref[...], w_ref[...],
                     preferred_element_type=jnp.float32)
        d = d_ref[0] + d_ref[1]
        deg = d[:_N, :1] + 1.0
        dinv = lax.rsqrt(deg)
        di_ref[...] = dinv
        u_ref[:_N, :] = p1 * dinv
        u_ref[_N:, :] = jnp.zeros((_NPAD - _N, _DH), jnp.float32)

    return pl.pallas_call(
        body,
        out_shape=(jax.ShapeDtypeStruct((_NPAD, _DH), jnp.float32),
                   jax.ShapeDtypeStruct((_N, 1), jnp.float32)),
    )(x, W1, degp)


def _mm2(s1p, dinv, b1, W2):
    def body(s_ref, di_ref, b_ref, w_ref, u_ref):
        s = s_ref[0, :_N, :] + s_ref[1, :_N, :]
        di = di_ref[...]
        h1 = jnp.maximum(s * di + b_ref[...], 0.0)
        u_ref[:_N, :] = jnp.dot(h1, w_ref[...],
                                preferred_element_type=jnp.float32) * di
        u_ref[_N:, :] = jnp.zeros((_NPAD - _N, _DH), jnp.float32)

    return pl.pallas_call(
        body,
        out_shape=jax.ShapeDtypeStruct((_NPAD, _DH), jnp.float32),
    )(s1p, dinv, b1, W2)


def _pool(s2p, dinv, b2, batch2d, Wl, bl):
    def body(s_ref, di_ref, b_ref, bat_ref, wl_ref, bl_ref, o_ref):
        s = s_ref[0, :_N, :] + s_ref[1, :_N, :]
        h2 = jnp.maximum(s * di_ref[...] + b_ref[...], 0.0)
        gids = lax.broadcasted_iota(jnp.int32, (_G, _N), 0)
        onehot = (bat_ref[...] == gids).astype(jnp.float32)
        summed = jnp.dot(onehot, h2, preferred_element_type=jnp.float32)
        cnt = jnp.sum(onehot, axis=1, keepdims=True)
        pooled = summed / jnp.maximum(cnt, 1.0)
        o_ref[...] = jnp.dot(pooled, wl_ref[...],
                             preferred_element_type=jnp.float32) + bl_ref[...]

    return pl.pallas_call(
        body,
        out_shape=jax.ShapeDtypeStruct((_G, _NCLS), jnp.float32),
    )(s2p, dinv, b2, batch2d, Wl, bl)


def kernel(x, edge_index, batch, W1, b1, W2, b2, Wl, bl):
    src = edge_index[0].reshape(_NW, _NCHUNK, _CH)
    dst = edge_index[1].reshape(_NW, _NCHUNK, _CH)
    ones_rows = jnp.ones((_CH, _DEGW), jnp.float32)
    zeros_deg = jnp.zeros((_NPAD, _DEGW), jnp.float32)
    zeros_u = jnp.zeros((_NPAD, _DH), jnp.float32)

    degp = _deg_kernel(dst, ones_rows, zeros_deg)
    u1, dinv = _mm1(x, W1, degp)
    s1p = _scatter_kernel(u1, src, dst, zeros_u)
    u2 = _mm2(s1p, dinv, b1.reshape(1, _DH), W2)
    s2p = _scatter_kernel(u2, src, dst, zeros_u)
    out = _pool(s2p, dinv, b2.reshape(1, _DH), batch.reshape(1, _N),
                Wl, bl.reshape(1, _NCLS))
    return out
